# baseline (speedup 1.0000x reference)
import functools

import jax
import jax.numpy as jnp
from jax import lax
from jax.experimental import pallas as pl
from jax.experimental.pallas import tpu as pltpu
from jax.experimental.pallas import tpu_sc as plsc

_NC = 2
_NS = 16
_CHUNK = 40


_NROW = 7
_GA = 4
_SW = 3
_NIDX = 14
_XA = 11


def _segment_sum_partials(x, edges3):
    n, d = x.shape
    nchunk = edges3.shape[0] // (_NC * _NS)
    rpt = (n // _NS) // 8 * 8
    tail0 = _NS * rpt
    ntail = n - tail0

    mesh = plsc.VectorSubcoreMesh(core_axis_name="c", subcore_axis_name="s")

    @functools.partial(
        pl.kernel,
        out_type=(
            jax.ShapeDtypeStruct((n, d), jnp.float32),
            jax.ShapeDtypeStruct((n, d), jnp.float32),
        ),
        mesh=mesh,
        scratch_types=[
            pltpu.VMEM((_NIDX, 2, _CHUNK), jnp.int32),
            pltpu.VMEM((_NROW, _CHUNK, d), jnp.float32),
            pltpu.VMEM_SHARED((n, d), jnp.float32),
        ] + [pltpu.SemaphoreType.DMA] * (2 * _NROW + _NIDX + 1),
    )
    def sc_kernel(x_hbm, e_hbm, p0_hbm, p1_hbm,
                  idx_v, rows_v, acc, *sems):
        gsems = sems[:_NROW]
        ssems = sems[_NROW:2 * _NROW]
        xsems = sems[2 * _NROW:2 * _NROW + _NIDX]
        isem = sems[-1]
        c = lax.axis_index("c")
        s = lax.axis_index("s")
        wid = c * _NS + s
        chunk0 = wid * nchunk
        row0 = pl.multiple_of(s * rpt, 8)

        def fire_idx(j, bi):
            pltpu.async_copy(e_hbm.at[chunk0 + j], idx_v.at[bi], xsems[bi])

        def fire_gather(j, br, bi):
            pltpu.make_async_copy(
                e_hbm.at[chunk0 + j], idx_v.at[bi], xsems[bi]).wait()
            pltpu.async_copy(x_hbm.at[idx_v.at[bi].at[0]],
                             rows_v.at[br], gsems[br])

        init_cp = pltpu.async_copy(
            x_hbm.at[pl.ds(row0, rpt)], acc.at[pl.ds(row0, rpt)], isem)
        for j in range(_XA):
            fire_idx(j, j)
        for j in range(_GA):
            fire_gather(j, j, j)
        init_cp.wait()
        if ntail:
            @pl.when(s == 0)
            def _():
                pltpu.sync_copy(x_hbm.at[pl.ds(tail0, ntail)],
                                acc.at[pl.ds(tail0, ntail)])
        plsc.subcore_barrier()

        def body(k, br, brn, bi, bin_, bix, static):
            pltpu.make_async_copy(
                x_hbm.at[idx_v.at[bi].at[0]], rows_v.at[br],
                gsems[br]).wait()
            pltpu.async_copy(rows_v.at[br], acc.at[idx_v.at[bi].at[1]],
                             ssems[br], add=True)
            if not static or k >= _SW:
                pltpu.make_async_copy(
                    rows_v.at[brn], acc.at[idx_v.at[bix].at[1]],
                    ssems[brn]).wait()
            if not static or k + _XA < nchunk:
                fire_idx(k + _XA, bix)
            if not static or k + _GA < nchunk:
                fire_gather(k + _GA, brn, bin_)

        for k in range(_NIDX):
            body(k, k % _NROW, (k + _GA) % _NROW, k % _NIDX,
                 (k + _GA) % _NIDX, (k + _XA) % _NIDX, static=True)

        main_end = _NIDX + (nchunk - _XA - _NIDX) // _NIDX * _NIDX

        @pl.loop(_NIDX, main_end, step=_NIDX)
        def _(g):
            for u in range(_NIDX):
                body(g + u, u % _NROW, (u + _GA) % _NROW, u,
                     (u + _GA) % _NIDX, (u + _XA) % _NIDX, static=False)

        for k in range(main_end, nchunk):
            body(k, k % _NROW, (k + _GA) % _NROW, k % _NIDX,
                 (k + _GA) % _NIDX, (k + _XA) % _NIDX, static=True)

        for j in range(nchunk - _SW, nchunk):
            pltpu.make_async_copy(
                rows_v.at[j % _NROW], acc.at[idx_v.at[j % _NIDX].at[1]],
                ssems[j % _NROW]).wait()

        plsc.subcore_barrier()

        @pl.when(c == 0)
        def _():
            pltpu.sync_copy(acc.at[pl.ds(row0, rpt)],
                            p0_hbm.at[pl.ds(row0, rpt)])
            if ntail:
                @pl.when(s == 0)
                def _():
                    pltpu.sync_copy(acc.at[pl.ds(tail0, ntail)],
                                    p0_hbm.at[pl.ds(tail0, ntail)])

        @pl.when(c == 1)
        def _():
            pltpu.sync_copy(acc.at[pl.ds(row0, rpt)],
                            p1_hbm.at[pl.ds(row0, rpt)])
            if ntail:
                @pl.when(s == 0)
                def _():
                    pltpu.sync_copy(acc.at[pl.ds(tail0, ntail)],
                                    p1_hbm.at[pl.ds(tail0, ntail)])

    return sc_kernel(x, edges3)


def _mlp(p0, p1, xin, wa, ba, wb, bb, relu_out):
    n, d = xin.shape
    o = wb.shape[1]
    br = 1000

    def body(p0_ref, p1_ref, x_ref, wa_ref, ba_ref, wb_ref, bb_ref, o_ref):
        hin = p0_ref[...] + p1_ref[...] - x_ref[...]
        h = jnp.dot(hin, wa_ref[...], preferred_element_type=jnp.float32)
        h = jnp.maximum(h + ba_ref[...], 0.0)
        h = jnp.dot(h, wb_ref[...], preferred_element_type=jnp.float32)
        h = h + bb_ref[...]
        if relu_out:
            h = jnp.maximum(h, 0.0)
        o_ref[...] = h

    return pl.pallas_call(
        body,
        grid=(n // br,),
        in_specs=[
            pl.BlockSpec((br, d), lambda i: (i, 0)),
            pl.BlockSpec((br, d), lambda i: (i, 0)),
            pl.BlockSpec((br, d), lambda i: (i, 0)),
            pl.BlockSpec((d, wa.shape[1]), lambda i: (0, 0)),
            pl.BlockSpec((1, wa.shape[1]), lambda i: (0, 0)),
            pl.BlockSpec((wb.shape[0], o), lambda i: (0, 0)),
            pl.BlockSpec((1, o), lambda i: (0, 0)),
        ],
        out_specs=pl.BlockSpec((br, o), lambda i: (i, 0)),
        out_shape=jax.ShapeDtypeStruct((n, o), jnp.float32),
    )(p0, p1, xin, wa, ba.reshape(1, -1), wb, bb.reshape(1, -1))


def kernel(x, edge_index, W1a, b1a, W1b, b1b, W2a, b2a, W2b, b2b):
    e = edge_index.shape[1]
    nck = e // _CHUNK
    edges3 = jnp.stack(
        [edge_index[0].astype(jnp.int32).reshape(nck, _CHUNK),
         edge_index[1].astype(jnp.int32).reshape(nck, _CHUNK)], axis=1)

    p0, p1 = _segment_sum_partials(x, edges3)
    h1 = _mlp(p0, p1, x, W1a, b1a, W1b, b1b, relu_out=True)

    q0, q1 = _segment_sum_partials(h1, edges3)
    out = _mlp(q0, q1, h1, W2a, b2a, W2b, b2b, relu_out=False)
    return out
